# baseline (speedup 1.0000x reference)
import functools

import jax
import jax.numpy as jnp
from jax import lax
from jax.experimental import pallas as pl
from jax.experimental.pallas import tpu as pltpu
from jax.experimental.pallas import tpu_sc as plsc

_N = 8192
_LAT = 256
_DIM = 512
_NCB = 4096

_BN = 1024
_BK = 512
_NB = _N // _BN
_KB = _NCB // _BK

_NC = 2
_NS = 16
_NW = _NC * _NS
_BPW = _N // _NW
_CH = 128
_NCHUNK = _BPW // _CH


def _vq_tc_body(z_ref, cb_ref, cnorm_ref, idx_ref, dsum_ref,
                runmin_ref, runidx_ref, acc_ref):
    i = pl.program_id(0)
    j = pl.program_id(1)
    z = z_ref[...]
    cb = cb_ref[...]
    cnorm = cnorm_ref[...]
    znorm = jnp.sum(z * z, axis=1, keepdims=True)
    mm = lax.dot_general(z, cb, (((1,), (1,)), ((), ())),
                         preferred_element_type=jnp.float32)
    d = (znorm + cnorm) - 2.0 * mm
    lmin = jnp.min(d, axis=1, keepdims=True)
    iota = lax.broadcasted_iota(jnp.int32, (_BN, _BK), 1)
    larg = jnp.min(jnp.where(d == lmin, iota, _NCB), axis=1,
                   keepdims=True) + j * _BK

    @pl.when(j == 0)
    def _():
        runmin_ref[...] = lmin
        runidx_ref[...] = larg

    @pl.when(j > 0)
    def _():
        prev_min = runmin_ref[...]
        prev_idx = runidx_ref[...]
        better = lmin < prev_min
        runmin_ref[...] = jnp.where(better, lmin, prev_min)
        runidx_ref[...] = jnp.where(better, larg, prev_idx)

    @pl.when(j == _KB - 1)
    def _():
        idx_ref[...] = runidx_ref[...]
        part = jnp.sum(runmin_ref[...])
        prev = jnp.where(i == 0, 0.0, acc_ref[0])
        tot = prev + part
        acc_ref[0] = tot
        dsum_ref[...] = jnp.reshape(tot, (1, 1))


def _vq_argmin(z_flat, cb, cnorm2d):
    idx2d, dsum = pl.pallas_call(
        _vq_tc_body,
        grid=(_NB, _KB),
        in_specs=[
            pl.BlockSpec((_BN, _DIM), lambda i, j: (i, 0)),
            pl.BlockSpec((_BK, _DIM), lambda i, j: (j, 0)),
            pl.BlockSpec((1, _BK), lambda i, j: (0, j)),
        ],
        out_specs=[
            pl.BlockSpec((_BN, 1), lambda i, j: (i, 0)),
            pl.BlockSpec((1, 1), lambda i, j: (0, 0)),
        ],
        out_shape=[
            jax.ShapeDtypeStruct((_N, 1), jnp.int32),
            jax.ShapeDtypeStruct((1, 1), jnp.float32),
        ],
        scratch_shapes=[
            pltpu.VMEM((_BN, 1), jnp.float32),
            pltpu.VMEM((_BN, 1), jnp.int32),
            pltpu.SMEM((1,), jnp.float32),
        ],
        compiler_params=pltpu.CompilerParams(
            dimension_semantics=("arbitrary", "arbitrary"),
        ),
    )(z_flat, cb, cnorm2d)
    return idx2d[:, 0], dsum[0, 0]


def _sc_gather(cb_syn, idx_syn, cb_sem, idx_sem):
    mesh = plsc.VectorSubcoreMesh(core_axis_name="c", subcore_axis_name="s")

    @functools.partial(
        pl.kernel,
        out_type=[
            jax.ShapeDtypeStruct((_N, _DIM), jnp.float32),
            jax.ShapeDtypeStruct((_N, _DIM), jnp.float32),
        ],
        mesh=mesh,
        scratch_types=[
            pltpu.VMEM((_CH,), jnp.int32),
            pltpu.VMEM((_CH, _DIM), jnp.float32),
            pltpu.SemaphoreType.DMA,
        ],
    )
    def k(cbs_hbm, idxs_hbm, cbm_hbm, idxm_hbm, outs_hbm, outm_hbm,
          idx_v, rows_v, sem):
        wid = lax.axis_index("s") * _NC + lax.axis_index("c")
        base = wid * _BPW
        for cb_h, idx_h, out_h in ((cbs_hbm, idxs_hbm, outs_hbm),
                                   (cbm_hbm, idxm_hbm, outm_hbm)):
            for c in range(_NCHUNK):
                off = base + c * _CH
                pltpu.sync_copy(idx_h.at[pl.ds(off, _CH)], idx_v)
                pltpu.async_copy(cb_h.at[idx_v], rows_v, sem).wait()
                pltpu.sync_copy(rows_v, out_h.at[pl.ds(off, _CH)])

    return k(cb_syn, idx_syn, cb_sem, idx_sem)


def kernel(z_fast_real, z_fast_imag, z_slow_real, z_slow_imag, cb_syn, cb_sem):
    zf = jnp.concatenate([z_fast_real, z_fast_imag], axis=-1)
    zs = jnp.concatenate([z_slow_real, z_slow_imag], axis=-1)
    cns = jnp.sum(cb_syn ** 2, axis=1)[None, :]
    cnm = jnp.sum(cb_sem ** 2, axis=1)[None, :]
    idx_syn, dsum_syn = _vq_argmin(zf, cb_syn, cns)
    idx_sem, dsum_sem = _vq_argmin(zs, cb_sem, cnm)
    rows_syn, rows_sem = _sc_gather(cb_syn, idx_syn, cb_sem, idx_sem)
    loss = 1.25 * (dsum_syn + dsum_sem) / (_N * _DIM)
    zq_syn = lax.complex(rows_syn[:, :_LAT], rows_syn[:, _LAT:])
    zq_sem = lax.complex(rows_sem[:, :_LAT], rows_sem[:, _LAT:])
    return (zq_syn, zq_sem, loss, idx_syn, idx_sem)


# speedup vs baseline: 1.0387x; 1.0387x over previous
import functools

import jax
import jax.numpy as jnp
from jax import lax
from jax.experimental import pallas as pl
from jax.experimental.pallas import tpu as pltpu
from jax.experimental.pallas import tpu_sc as plsc

_N = 8192
_LAT = 256
_DIM = 512
_NCB = 4096

_BN = 1024
_BK = 512
_NB = _N // _BN
_KB = _NCB // _BK

_NC = 2
_NS = 16
_NW = _NC * _NS
_BPW = _N // _NW
_CH = 128
_NCHUNK = _BPW // _CH


def _epilogue(mm, znorm, cnorm, jj):
    d = (znorm + cnorm) - 2.0 * mm
    lmin = jnp.min(d, axis=1, keepdims=True)
    iota = lax.broadcasted_iota(jnp.int32, (_BN, _BK), 1)
    larg = jnp.min(jnp.where(d == lmin, iota, _NCB), axis=1,
                   keepdims=True) + jj * _BK
    return lmin, larg


def _vq_tc_body(z_ref, znorm_ref, cb_ref, cnorm_prev_ref, cnorm_cur_ref,
                idx_ref, dsum_ref, mm_ref, runmin_ref, runidx_ref, acc_ref):
    i = pl.program_id(0)
    j = pl.program_id(1)
    znorm = znorm_ref[...]

    lmin, larg = _epilogue(mm_ref[...], znorm, cnorm_prev_ref[...], j - 1)
    better = (lmin < runmin_ref[...]) | (j == 1)
    runmin_ref[...] = jnp.where(better, lmin, runmin_ref[...])
    runidx_ref[...] = jnp.where(better, larg, runidx_ref[...])

    mm_ref[...] = lax.dot_general(
        z_ref[...], cb_ref[...], (((1,), (1,)), ((), ())),
        preferred_element_type=jnp.float32)

    @pl.when(j == _KB - 1)
    def _():
        lmin2, larg2 = _epilogue(mm_ref[...], znorm, cnorm_cur_ref[...], j)
        better2 = lmin2 < runmin_ref[...]
        fmin = jnp.where(better2, lmin2, runmin_ref[...])
        idx_ref[...] = jnp.where(better2, larg2, runidx_ref[...])
        part = jnp.sum(fmin)
        prev = jnp.where(i == 0, 0.0, acc_ref[0])
        tot = prev + part
        acc_ref[0] = tot
        dsum_ref[...] = jnp.reshape(tot, (1, 1))


def _vq_argmin(z_flat, znorm2d, cb, cnorm2d):
    idx2d, dsum = pl.pallas_call(
        _vq_tc_body,
        grid=(_NB, _KB),
        in_specs=[
            pl.BlockSpec((_BN, _DIM), lambda i, j: (i, 0)),
            pl.BlockSpec((_BN, 1), lambda i, j: (i, 0)),
            pl.BlockSpec((_BK, _DIM), lambda i, j: (j, 0)),
            pl.BlockSpec((1, _BK), lambda i, j: (0, jnp.maximum(j - 1, 0))),
            pl.BlockSpec((1, _BK), lambda i, j: (0, j)),
        ],
        out_specs=[
            pl.BlockSpec((_BN, 1), lambda i, j: (i, 0)),
            pl.BlockSpec((1, 1), lambda i, j: (0, 0)),
        ],
        out_shape=[
            jax.ShapeDtypeStruct((_N, 1), jnp.int32),
            jax.ShapeDtypeStruct((1, 1), jnp.float32),
        ],
        scratch_shapes=[
            pltpu.VMEM((_BN, _BK), jnp.float32),
            pltpu.VMEM((_BN, 1), jnp.float32),
            pltpu.VMEM((_BN, 1), jnp.int32),
            pltpu.SMEM((1,), jnp.float32),
        ],
        compiler_params=pltpu.CompilerParams(
            dimension_semantics=("arbitrary", "arbitrary"),
        ),
    )(z_flat, znorm2d, cb, cnorm2d, cnorm2d)
    return idx2d[:, 0], dsum[0, 0]


def _sc_gather_one(cb, idx):
    mesh = plsc.VectorSubcoreMesh(core_axis_name="c", subcore_axis_name="s")

    @functools.partial(
        pl.kernel,
        out_type=jax.ShapeDtypeStruct((_N, _DIM), jnp.float32),
        mesh=mesh,
        scratch_types=[
            pltpu.VMEM((_CH,), jnp.int32),
            pltpu.VMEM((_CH, _DIM), jnp.float32),
            pltpu.SemaphoreType.DMA,
        ],
    )
    def k(cb_hbm, idx_hbm, out_hbm, idx_v, rows_v, sem):
        wid = lax.axis_index("s") * _NC + lax.axis_index("c")
        base = wid * _BPW
        for c in range(_NCHUNK):
            off = base + c * _CH
            pltpu.sync_copy(idx_hbm.at[pl.ds(off, _CH)], idx_v)
            pltpu.async_copy(cb_hbm.at[idx_v], rows_v, sem).wait()
            pltpu.sync_copy(rows_v, out_hbm.at[pl.ds(off, _CH)])

    return k(cb, idx)


def kernel(z_fast_real, z_fast_imag, z_slow_real, z_slow_imag, cb_syn, cb_sem):
    zf = jnp.concatenate([z_fast_real, z_fast_imag], axis=-1)
    zs = jnp.concatenate([z_slow_real, z_slow_imag], axis=-1)
    znf = jnp.sum(zf ** 2, axis=1)[:, None]
    zns = jnp.sum(zs ** 2, axis=1)[:, None]
    cns = jnp.sum(cb_syn ** 2, axis=1)[None, :]
    cnm = jnp.sum(cb_sem ** 2, axis=1)[None, :]
    idx_syn, dsum_syn = _vq_argmin(zf, znf, cb_syn, cns)
    rows_syn = _sc_gather_one(cb_syn, idx_syn)
    idx_sem, dsum_sem = _vq_argmin(zs, zns, cb_sem, cnm)
    rows_sem = _sc_gather_one(cb_sem, idx_sem)
    loss = 1.25 * (dsum_syn + dsum_sem) / (_N * _DIM)
    zq_syn = lax.complex(rows_syn[:, :_LAT], rows_syn[:, _LAT:])
    zq_sem = lax.complex(rows_sem[:, :_LAT], rows_sem[:, _LAT:])
    return (zq_syn, zq_sem, loss, idx_syn, idx_sem)


# speedup vs baseline: 1.0621x; 1.0226x over previous
import functools

import jax
import jax.numpy as jnp
from jax import lax
from jax.experimental import pallas as pl
from jax.experimental.pallas import tpu as pltpu
from jax.experimental.pallas import tpu_sc as plsc

_N = 8192
_LAT = 256
_DIM = 512
_NCB = 4096

_BN = 1024
_BK = 512
_NB = _N // _BN
_KB = _NCB // _BK

_NC = 2
_NS = 16
_NW = _NC * _NS
_BPW = _N // _NW
_CH = 128
_NCHUNK = _BPW // _CH


def _epilogue(mm, znorm, cnorm, jj):
    d = (znorm + cnorm) - 2.0 * mm
    lmin = jnp.min(d, axis=1, keepdims=True)
    iota = lax.broadcasted_iota(jnp.int32, (_BN, _BK), 1)
    larg = jnp.min(jnp.where(d == lmin, iota, _NCB), axis=1,
                   keepdims=True) + jj * _BK
    return lmin, larg


def _vq_tc_body(z_ref, znorm_ref, cb_ref, cnorm_prev_ref, cnorm_cur_ref,
                idx_ref, dsum_ref, mm_ref, runmin_ref, runidx_ref, acc_ref):
    i = pl.program_id(0)
    j = pl.program_id(1)
    znorm = znorm_ref[...]

    lmin, larg = _epilogue(mm_ref[...], znorm, cnorm_prev_ref[...], j - 1)
    better = (lmin < runmin_ref[...]) | (j == 1)
    runmin_ref[...] = jnp.where(better, lmin, runmin_ref[...])
    runidx_ref[...] = jnp.where(better, larg, runidx_ref[...])

    mm_ref[...] = lax.dot_general(
        z_ref[...], cb_ref[...], (((1,), (1,)), ((), ())),
        preferred_element_type=jnp.float32)

    @pl.when(j == _KB - 1)
    def _():
        lmin2, larg2 = _epilogue(mm_ref[...], znorm, cnorm_cur_ref[...], j)
        better2 = lmin2 < runmin_ref[...]
        fmin = jnp.where(better2, lmin2, runmin_ref[...])
        idx_ref[...] = jnp.where(better2, larg2, runidx_ref[...])
        part = jnp.sum(fmin)
        prev = jnp.where(i == 0, 0.0, acc_ref[0])
        tot = prev + part
        acc_ref[0] = tot
        dsum_ref[...] = jnp.reshape(tot, (1, 1))


def _vq_argmin(z_flat, znorm2d, cb, cnorm2d):
    idx2d, dsum = pl.pallas_call(
        _vq_tc_body,
        grid=(_NB, _KB),
        in_specs=[
            pl.BlockSpec((_BN, _DIM), lambda i, j: (i, 0)),
            pl.BlockSpec((_BN, 1), lambda i, j: (i, 0)),
            pl.BlockSpec((_BK, _DIM), lambda i, j: (j, 0)),
            pl.BlockSpec((1, _BK), lambda i, j: (0, jnp.maximum(j - 1, 0))),
            pl.BlockSpec((1, _BK), lambda i, j: (0, j)),
        ],
        out_specs=[
            pl.BlockSpec((_BN, 1), lambda i, j: (i, 0)),
            pl.BlockSpec((1, 1), lambda i, j: (0, 0)),
        ],
        out_shape=[
            jax.ShapeDtypeStruct((_N, 1), jnp.int32),
            jax.ShapeDtypeStruct((1, 1), jnp.float32),
        ],
        scratch_shapes=[
            pltpu.VMEM((_BN, _BK), jnp.float32),
            pltpu.VMEM((_BN, 1), jnp.float32),
            pltpu.VMEM((_BN, 1), jnp.int32),
            pltpu.SMEM((1,), jnp.float32),
        ],
        compiler_params=pltpu.CompilerParams(
            dimension_semantics=("arbitrary", "arbitrary"),
        ),
    )(z_flat, znorm2d, cb, cnorm2d, cnorm2d)
    return idx2d[:, 0], dsum[0, 0]


def _sc_gather_one(cb_r, cb_i, idx):
    mesh = plsc.VectorSubcoreMesh(core_axis_name="c", subcore_axis_name="s")

    @functools.partial(
        pl.kernel,
        out_type=[
            jax.ShapeDtypeStruct((_N, _LAT), jnp.float32),
            jax.ShapeDtypeStruct((_N, _LAT), jnp.float32),
        ],
        mesh=mesh,
        scratch_types=[
            pltpu.VMEM((_CH,), jnp.int32),
            pltpu.VMEM((_CH, _LAT), jnp.float32),
            pltpu.VMEM((_CH, _LAT), jnp.float32),
            pltpu.SemaphoreType.DMA,
        ],
    )
    def k(cbr_hbm, cbi_hbm, idx_hbm, outr_hbm, outi_hbm,
          idx_v, rows_r, rows_i, sem):
        wid = lax.axis_index("s") * _NC + lax.axis_index("c")
        base = wid * _BPW
        for c in range(_NCHUNK):
            off = base + c * _CH
            pltpu.sync_copy(idx_hbm.at[pl.ds(off, _CH)], idx_v)
            cp_r = pltpu.async_copy(cbr_hbm.at[idx_v], rows_r, sem)
            cp_i = pltpu.async_copy(cbi_hbm.at[idx_v], rows_i, sem)
            cp_r.wait()
            cp_i.wait()
            pltpu.sync_copy(rows_r, outr_hbm.at[pl.ds(off, _CH)])
            pltpu.sync_copy(rows_i, outi_hbm.at[pl.ds(off, _CH)])

    return k(cb_r, cb_i, idx)


def kernel(z_fast_real, z_fast_imag, z_slow_real, z_slow_imag, cb_syn, cb_sem):
    zf = jnp.concatenate([z_fast_real, z_fast_imag], axis=-1)
    zs = jnp.concatenate([z_slow_real, z_slow_imag], axis=-1)
    znf = jnp.sum(zf ** 2, axis=1)[:, None]
    zns = jnp.sum(zs ** 2, axis=1)[:, None]
    cns = jnp.sum(cb_syn ** 2, axis=1)[None, :]
    cnm = jnp.sum(cb_sem ** 2, axis=1)[None, :]
    idx_syn, dsum_syn = _vq_argmin(zf, znf, cb_syn, cns)
    rs_r, rs_i = _sc_gather_one(cb_syn[:, :_LAT], cb_syn[:, _LAT:], idx_syn)
    idx_sem, dsum_sem = _vq_argmin(zs, zns, cb_sem, cnm)
    rm_r, rm_i = _sc_gather_one(cb_sem[:, :_LAT], cb_sem[:, _LAT:], idx_sem)
    loss = 1.25 * (dsum_syn + dsum_sem) / (_N * _DIM)
    zq_syn = lax.complex(rs_r, rs_i)
    zq_sem = lax.complex(rm_r, rm_i)
    return (zq_syn, zq_sem, loss, idx_syn, idx_sem)


# speedup vs baseline: 1.0731x; 1.0103x over previous
import functools

import jax
import jax.numpy as jnp
from jax import lax
from jax.experimental import pallas as pl
from jax.experimental.pallas import tpu as pltpu
from jax.experimental.pallas import tpu_sc as plsc

_N = 8192
_LAT = 256
_DIM = 512
_NCB = 4096

_BN = 1024
_BK = 512
_NB = _N // _BN
_KB = _NCB // _BK

_NC = 2
_NS = 16
_NW = _NC * _NS
_BPW = _N // _NW
_CH = 128
_NCHUNK = _BPW // _CH


def _epilogue(mm, znorm, cnorm, iota_row, jjf):
    d = (znorm + cnorm) - 2.0 * mm
    lmin = jnp.min(d, axis=1, keepdims=True)
    cand = jnp.where(d == lmin, iota_row, float(_NCB))
    larg = jnp.min(cand, axis=1, keepdims=True) + jjf
    return lmin, larg


def _vq_tc_body(z_ref, znorm_ref, cb_ref, cnorm_prev_ref, cnorm_cur_ref,
                iota_ref, idx_ref, dsum_ref, mm_ref, runmin_ref, runidx_ref,
                acc_ref):
    i = pl.program_id(0)
    j = pl.program_id(1)
    znorm = znorm_ref[...]
    iota_row = iota_ref[...]

    jjf = ((j - 1) * _BK).astype(jnp.float32)
    lmin, larg = _epilogue(mm_ref[...], znorm, cnorm_prev_ref[...],
                           iota_row, jjf)
    better = (lmin < runmin_ref[...]) | (j == 1)
    runmin_ref[...] = jnp.where(better, lmin, runmin_ref[...])
    runidx_ref[...] = jnp.where(better, larg, runidx_ref[...])

    mm_ref[...] = lax.dot_general(
        z_ref[...], cb_ref[...], (((1,), (1,)), ((), ())),
        preferred_element_type=jnp.float32)

    @pl.when(j == _KB - 1)
    def _():
        lmin2, larg2 = _epilogue(mm_ref[...], znorm, cnorm_cur_ref[...],
                                 iota_row, jnp.float32(j * _BK))
        better2 = lmin2 < runmin_ref[...]
        fmin = jnp.where(better2, lmin2, runmin_ref[...])
        idx_ref[...] = jnp.where(better2, larg2,
                                 runidx_ref[...]).astype(jnp.int32)
        part = jnp.sum(fmin)
        prev = jnp.where(i == 0, 0.0, acc_ref[0])
        tot = prev + part
        acc_ref[0] = tot
        dsum_ref[...] = jnp.reshape(tot, (1, 1))


def _vq_argmin(z_flat, znorm2d, cb, cnorm2d):
    idx2d, dsum = pl.pallas_call(
        _vq_tc_body,
        grid=(_NB, _KB),
        in_specs=[
            pl.BlockSpec((_BN, _DIM), lambda i, j: (i, 0)),
            pl.BlockSpec((_BN, 1), lambda i, j: (i, 0)),
            pl.BlockSpec((_BK, _DIM), lambda i, j: (j, 0)),
            pl.BlockSpec((1, _BK), lambda i, j: (0, jnp.maximum(j - 1, 0))),
            pl.BlockSpec((1, _BK), lambda i, j: (0, j)),
            pl.BlockSpec((1, _BK), lambda i, j: (0, 0)),
        ],
        out_specs=[
            pl.BlockSpec((_BN, 1), lambda i, j: (i, 0)),
            pl.BlockSpec((1, 1), lambda i, j: (0, 0)),
        ],
        out_shape=[
            jax.ShapeDtypeStruct((_N, 1), jnp.int32),
            jax.ShapeDtypeStruct((1, 1), jnp.float32),
        ],
        scratch_shapes=[
            pltpu.VMEM((_BN, _BK), jnp.float32),
            pltpu.VMEM((_BN, 1), jnp.float32),
            pltpu.VMEM((_BN, 1), jnp.float32),
            pltpu.SMEM((1,), jnp.float32),
        ],
        compiler_params=pltpu.CompilerParams(
            dimension_semantics=("arbitrary", "arbitrary"),
        ),
    )(z_flat, znorm2d, cb, cnorm2d, cnorm2d,
      jnp.arange(_BK, dtype=jnp.float32)[None, :])
    return idx2d[:, 0], dsum[0, 0]


def _sc_gather_one(cb_r, cb_i, idx):
    mesh = plsc.VectorSubcoreMesh(core_axis_name="c", subcore_axis_name="s")

    @functools.partial(
        pl.kernel,
        out_type=[
            jax.ShapeDtypeStruct((_N, _LAT), jnp.float32),
            jax.ShapeDtypeStruct((_N, _LAT), jnp.float32),
        ],
        mesh=mesh,
        scratch_types=[
            pltpu.VMEM((_CH,), jnp.int32),
            pltpu.VMEM((_CH, _LAT), jnp.float32),
            pltpu.VMEM((_CH, _LAT), jnp.float32),
            pltpu.SemaphoreType.DMA,
        ],
    )
    def k(cbr_hbm, cbi_hbm, idx_hbm, outr_hbm, outi_hbm,
          idx_v, rows_r, rows_i, sem):
        wid = lax.axis_index("s") * _NC + lax.axis_index("c")
        base = wid * _BPW
        for c in range(_NCHUNK):
            off = base + c * _CH
            pltpu.sync_copy(idx_hbm.at[pl.ds(off, _CH)], idx_v)
            cp_r = pltpu.async_copy(cbr_hbm.at[idx_v], rows_r, sem)
            cp_i = pltpu.async_copy(cbi_hbm.at[idx_v], rows_i, sem)
            cp_r.wait()
            cp_i.wait()
            pltpu.sync_copy(rows_r, outr_hbm.at[pl.ds(off, _CH)])
            pltpu.sync_copy(rows_i, outi_hbm.at[pl.ds(off, _CH)])

    return k(cb_r, cb_i, idx)


def kernel(z_fast_real, z_fast_imag, z_slow_real, z_slow_imag, cb_syn, cb_sem):
    zf = jnp.concatenate([z_fast_real, z_fast_imag], axis=-1)
    zs = jnp.concatenate([z_slow_real, z_slow_imag], axis=-1)
    znf = jnp.sum(zf ** 2, axis=1)[:, None]
    zns = jnp.sum(zs ** 2, axis=1)[:, None]
    cns = jnp.sum(cb_syn ** 2, axis=1)[None, :]
    cnm = jnp.sum(cb_sem ** 2, axis=1)[None, :]
    idx_syn, dsum_syn = _vq_argmin(zf, znf, cb_syn, cns)
    rs_r, rs_i = _sc_gather_one(cb_syn[:, :_LAT], cb_syn[:, _LAT:], idx_syn)
    idx_sem, dsum_sem = _vq_argmin(zs, zns, cb_sem, cnm)
    rm_r, rm_i = _sc_gather_one(cb_sem[:, :_LAT], cb_sem[:, _LAT:], idx_sem)
    loss = 1.25 * (dsum_syn + dsum_sem) / (_N * _DIM)
    zq_syn = lax.complex(rs_r, rs_i)
    zq_sem = lax.complex(rm_r, rm_i)
    return (zq_syn, zq_sem, loss, idx_syn, idx_sem)


# speedup vs baseline: 1.1314x; 1.0544x over previous
import functools

import jax
import jax.numpy as jnp
from jax import lax
from jax.experimental import pallas as pl
from jax.experimental.pallas import tpu as pltpu
from jax.experimental.pallas import tpu_sc as plsc

_N = 8192
_LAT = 256
_DIM = 512
_NCB = 4096

_BN = 1024
_BK = 512
_NB = _N // _BN
_KB = _NCB // _BK

_NC = 2
_NS = 16
_NW = _NC * _NS
_BPW = _N // _NW
_CH = 128
_NCHUNK = _BPW // _CH


def _epilogue(mm, znorm, cnorm, iota_row, jjf):
    d = (znorm + cnorm) - 2.0 * mm
    lmin = jnp.min(d, axis=1, keepdims=True)
    cand = jnp.where(d == lmin, iota_row, float(_NCB))
    larg = jnp.min(cand, axis=1, keepdims=True) + jjf
    return lmin, larg


def _vq_tc_body(zr_ref, zi_ref, znorm_ref, cb_ref, cnorm_prev_ref,
                cnorm_cur_ref, iota_ref, idx_ref, dsum_ref, mm_ref,
                runmin_ref, runidx_ref, acc_ref):
    i = pl.program_id(0)
    j = pl.program_id(1)
    znorm = znorm_ref[...]
    iota_row = iota_ref[...]

    jjf = ((j - 1) * _BK).astype(jnp.float32)
    lmin, larg = _epilogue(mm_ref[...], znorm, cnorm_prev_ref[...],
                           iota_row, jjf)
    better = (lmin < runmin_ref[...]) | (j == 1)
    runmin_ref[...] = jnp.where(better, lmin, runmin_ref[...])
    runidx_ref[...] = jnp.where(better, larg, runidx_ref[...])

    cb = cb_ref[...]
    mm_ref[...] = (
        lax.dot_general(zr_ref[...], cb[:, :_LAT], (((1,), (1,)), ((), ())),
                        preferred_element_type=jnp.float32)
        + lax.dot_general(zi_ref[...], cb[:, _LAT:], (((1,), (1,)), ((), ())),
                          preferred_element_type=jnp.float32))

    @pl.when(j == _KB - 1)
    def _():
        lmin2, larg2 = _epilogue(mm_ref[...], znorm, cnorm_cur_ref[...],
                                 iota_row, jnp.float32(j * _BK))
        better2 = lmin2 < runmin_ref[...]
        fmin = jnp.where(better2, lmin2, runmin_ref[...])
        idx_ref[...] = jnp.where(better2, larg2,
                                 runidx_ref[...]).astype(jnp.int32)
        part = jnp.sum(fmin)
        prev = jnp.where(i == 0, 0.0, acc_ref[0])
        tot = prev + part
        acc_ref[0] = tot
        dsum_ref[...] = jnp.reshape(tot, (1, 1))


def _vq_argmin(z_r, z_i, znorm2d, cb, cnorm2d):
    idx2d, dsum = pl.pallas_call(
        _vq_tc_body,
        grid=(_NB, _KB),
        in_specs=[
            pl.BlockSpec((_BN, _LAT), lambda i, j: (i, 0)),
            pl.BlockSpec((_BN, _LAT), lambda i, j: (i, 0)),
            pl.BlockSpec((_BN, 1), lambda i, j: (i, 0)),
            pl.BlockSpec((_BK, _DIM), lambda i, j: (j, 0)),
            pl.BlockSpec((1, _BK), lambda i, j: (0, jnp.maximum(j - 1, 0))),
            pl.BlockSpec((1, _BK), lambda i, j: (0, j)),
            pl.BlockSpec((1, _BK), lambda i, j: (0, 0)),
        ],
        out_specs=[
            pl.BlockSpec((_BN, 1), lambda i, j: (i, 0)),
            pl.BlockSpec((1, 1), lambda i, j: (0, 0)),
        ],
        out_shape=[
            jax.ShapeDtypeStruct((_N, 1), jnp.int32),
            jax.ShapeDtypeStruct((1, 1), jnp.float32),
        ],
        scratch_shapes=[
            pltpu.VMEM((_BN, _BK), jnp.float32),
            pltpu.VMEM((_BN, 1), jnp.float32),
            pltpu.VMEM((_BN, 1), jnp.float32),
            pltpu.SMEM((1,), jnp.float32),
        ],
        compiler_params=pltpu.CompilerParams(
            dimension_semantics=("arbitrary", "arbitrary"),
        ),
    )(z_r, z_i, znorm2d, cb, cnorm2d, cnorm2d,
      jnp.arange(_BK, dtype=jnp.float32)[None, :])
    return idx2d[:, 0], dsum[0, 0]


def _sc_gather_one(cb_r, cb_i, idx):
    mesh = plsc.VectorSubcoreMesh(core_axis_name="c", subcore_axis_name="s")

    @functools.partial(
        pl.kernel,
        out_type=[
            jax.ShapeDtypeStruct((_N, _LAT), jnp.float32),
            jax.ShapeDtypeStruct((_N, _LAT), jnp.float32),
        ],
        mesh=mesh,
        scratch_types=[
            pltpu.VMEM((_CH,), jnp.int32),
            pltpu.VMEM((_CH, _LAT), jnp.float32),
            pltpu.VMEM((_CH, _LAT), jnp.float32),
            pltpu.SemaphoreType.DMA,
        ],
    )
    def k(cbr_hbm, cbi_hbm, idx_hbm, outr_hbm, outi_hbm,
          idx_v, rows_r, rows_i, sem):
        wid = lax.axis_index("s") * _NC + lax.axis_index("c")
        base = wid * _BPW
        for c in range(_NCHUNK):
            off = base + c * _CH
            pltpu.sync_copy(idx_hbm.at[pl.ds(off, _CH)], idx_v)
            cp_r = pltpu.async_copy(cbr_hbm.at[idx_v], rows_r, sem)
            cp_i = pltpu.async_copy(cbi_hbm.at[idx_v], rows_i, sem)
            cp_r.wait()
            cp_i.wait()
            pltpu.sync_copy(rows_r, outr_hbm.at[pl.ds(off, _CH)])
            pltpu.sync_copy(rows_i, outi_hbm.at[pl.ds(off, _CH)])

    return k(cb_r, cb_i, idx)


def kernel(z_fast_real, z_fast_imag, z_slow_real, z_slow_imag, cb_syn, cb_sem):
    zf = jnp.concatenate([z_fast_real, z_fast_imag], axis=-1)
    zs = jnp.concatenate([z_slow_real, z_slow_imag], axis=-1)
    znf = jnp.sum(zf ** 2, axis=1)[:, None]
    zns = jnp.sum(zs ** 2, axis=1)[:, None]
    cns = jnp.sum(cb_syn ** 2, axis=1)[None, :]
    cnm = jnp.sum(cb_sem ** 2, axis=1)[None, :]
    idx_syn, dsum_syn = _vq_argmin(z_fast_real, z_fast_imag, znf, cb_syn, cns)
    rs_r, rs_i = _sc_gather_one(cb_syn[:, :_LAT], cb_syn[:, _LAT:], idx_syn)
    idx_sem, dsum_sem = _vq_argmin(z_slow_real, z_slow_imag, zns, cb_sem, cnm)
    rm_r, rm_i = _sc_gather_one(cb_sem[:, :_LAT], cb_sem[:, _LAT:], idx_sem)
    loss = 1.25 * (dsum_syn + dsum_sem) / (_N * _DIM)
    zq_syn = lax.complex(rs_r, rs_i)
    zq_sem = lax.complex(rm_r, rm_i)
    return (zq_syn, zq_sem, loss, idx_syn, idx_sem)


# speedup vs baseline: 1.1713x; 1.0353x over previous
import functools

import jax
import jax.numpy as jnp
from jax import lax
from jax.experimental import pallas as pl
from jax.experimental.pallas import tpu as pltpu
from jax.experimental.pallas import tpu_sc as plsc

_N = 8192
_LAT = 256
_DIM = 512
_NCB = 4096

_BN = 2048
_BK = 512
_NB = _N // _BN
_KB = _NCB // _BK

_NC = 2
_NS = 16
_NW = _NC * _NS
_BPW = _N // _NW
_CH = 128
_NCHUNK = _BPW // _CH


def _epilogue(mm, znorm, cnorm, iota_row, jjf):
    d = (znorm + cnorm) - 2.0 * mm
    lmin = jnp.min(d, axis=1, keepdims=True)
    cand = jnp.where(d == lmin, iota_row, float(_NCB))
    larg = jnp.min(cand, axis=1, keepdims=True) + jjf
    return lmin, larg


def _vq_tc_body(zr_ref, zi_ref, znorm_ref, cb_ref, cnorm_prev_ref,
                cnorm_cur_ref, iota_ref, idx_ref, dsum_ref, mm_ref,
                runmin_ref, runidx_ref, acc_ref):
    i = pl.program_id(0)
    j = pl.program_id(1)
    znorm = znorm_ref[...]
    iota_row = iota_ref[...]

    jjf = ((j - 1) * _BK).astype(jnp.float32)
    lmin, larg = _epilogue(mm_ref[...], znorm, cnorm_prev_ref[...],
                           iota_row, jjf)
    better = (lmin < runmin_ref[...]) | (j == 1)
    runmin_ref[...] = jnp.where(better, lmin, runmin_ref[...])
    runidx_ref[...] = jnp.where(better, larg, runidx_ref[...])

    cb = cb_ref[...]
    mm_ref[...] = (
        lax.dot_general(zr_ref[...], cb[:, :_LAT], (((1,), (1,)), ((), ())),
                        preferred_element_type=jnp.float32)
        + lax.dot_general(zi_ref[...], cb[:, _LAT:], (((1,), (1,)), ((), ())),
                          preferred_element_type=jnp.float32))

    @pl.when(j == _KB - 1)
    def _():
        lmin2, larg2 = _epilogue(mm_ref[...], znorm, cnorm_cur_ref[...],
                                 iota_row, jnp.float32(j * _BK))
        better2 = lmin2 < runmin_ref[...]
        fmin = jnp.where(better2, lmin2, runmin_ref[...])
        idx_ref[...] = jnp.where(better2, larg2,
                                 runidx_ref[...]).astype(jnp.int32)
        part = jnp.sum(fmin)
        prev = jnp.where(i == 0, 0.0, acc_ref[0])
        tot = prev + part
        acc_ref[0] = tot
        dsum_ref[...] = jnp.reshape(tot, (1, 1))


def _vq_argmin(z_r, z_i, znorm2d, cb, cnorm2d):
    idx2d, dsum = pl.pallas_call(
        _vq_tc_body,
        grid=(_NB, _KB),
        in_specs=[
            pl.BlockSpec((_BN, _LAT), lambda i, j: (i, 0)),
            pl.BlockSpec((_BN, _LAT), lambda i, j: (i, 0)),
            pl.BlockSpec((_BN, 1), lambda i, j: (i, 0)),
            pl.BlockSpec((_BK, _DIM), lambda i, j: (j, 0)),
            pl.BlockSpec((1, _BK), lambda i, j: (0, jnp.maximum(j - 1, 0))),
            pl.BlockSpec((1, _BK), lambda i, j: (0, j)),
            pl.BlockSpec((1, _BK), lambda i, j: (0, 0)),
        ],
        out_specs=[
            pl.BlockSpec((_BN, 1), lambda i, j: (i, 0)),
            pl.BlockSpec((1, 1), lambda i, j: (0, 0)),
        ],
        out_shape=[
            jax.ShapeDtypeStruct((_N, 1), jnp.int32),
            jax.ShapeDtypeStruct((1, 1), jnp.float32),
        ],
        scratch_shapes=[
            pltpu.VMEM((_BN, _BK), jnp.float32),
            pltpu.VMEM((_BN, 1), jnp.float32),
            pltpu.VMEM((_BN, 1), jnp.float32),
            pltpu.SMEM((1,), jnp.float32),
        ],
        compiler_params=pltpu.CompilerParams(
            dimension_semantics=("arbitrary", "arbitrary"),
        ),
    )(z_r, z_i, znorm2d, cb, cnorm2d, cnorm2d,
      jnp.arange(_BK, dtype=jnp.float32)[None, :])
    return idx2d[:, 0], dsum[0, 0]


def _sc_gather_one(cb_r, cb_i, idx):
    mesh = plsc.VectorSubcoreMesh(core_axis_name="c", subcore_axis_name="s")

    @functools.partial(
        pl.kernel,
        out_type=[
            jax.ShapeDtypeStruct((_N, _LAT), jnp.float32),
            jax.ShapeDtypeStruct((_N, _LAT), jnp.float32),
        ],
        mesh=mesh,
        scratch_types=[
            pltpu.VMEM((_CH,), jnp.int32),
            pltpu.VMEM((_CH, _LAT), jnp.float32),
            pltpu.VMEM((_CH, _LAT), jnp.float32),
            pltpu.SemaphoreType.DMA,
        ],
    )
    def k(cbr_hbm, cbi_hbm, idx_hbm, outr_hbm, outi_hbm,
          idx_v, rows_r, rows_i, sem):
        wid = lax.axis_index("s") * _NC + lax.axis_index("c")
        base = wid * _BPW
        for c in range(_NCHUNK):
            off = base + c * _CH
            pltpu.sync_copy(idx_hbm.at[pl.ds(off, _CH)], idx_v)
            cp_r = pltpu.async_copy(cbr_hbm.at[idx_v], rows_r, sem)
            cp_i = pltpu.async_copy(cbi_hbm.at[idx_v], rows_i, sem)
            cp_r.wait()
            cp_i.wait()
            pltpu.sync_copy(rows_r, outr_hbm.at[pl.ds(off, _CH)])
            pltpu.sync_copy(rows_i, outi_hbm.at[pl.ds(off, _CH)])

    return k(cb_r, cb_i, idx)


def kernel(z_fast_real, z_fast_imag, z_slow_real, z_slow_imag, cb_syn, cb_sem):
    zf = jnp.concatenate([z_fast_real, z_fast_imag], axis=-1)
    zs = jnp.concatenate([z_slow_real, z_slow_imag], axis=-1)
    znf = jnp.sum(zf ** 2, axis=1)[:, None]
    zns = jnp.sum(zs ** 2, axis=1)[:, None]
    cns = jnp.sum(cb_syn ** 2, axis=1)[None, :]
    cnm = jnp.sum(cb_sem ** 2, axis=1)[None, :]
    idx_syn, dsum_syn = _vq_argmin(z_fast_real, z_fast_imag, znf, cb_syn, cns)
    rs_r, rs_i = _sc_gather_one(cb_syn[:, :_LAT], cb_syn[:, _LAT:], idx_syn)
    idx_sem, dsum_sem = _vq_argmin(z_slow_real, z_slow_imag, zns, cb_sem, cnm)
    rm_r, rm_i = _sc_gather_one(cb_sem[:, :_LAT], cb_sem[:, _LAT:], idx_sem)
    loss = 1.25 * (dsum_syn + dsum_sem) / (_N * _DIM)
    zq_syn = lax.complex(rs_r, rs_i)
    zq_sem = lax.complex(rm_r, rm_i)
    return (zq_syn, zq_sem, loss, idx_syn, idx_sem)


# speedup vs baseline: 1.1827x; 1.0097x over previous
import functools

import jax
import jax.numpy as jnp
from jax import lax
from jax.experimental import pallas as pl
from jax.experimental.pallas import tpu as pltpu
from jax.experimental.pallas import tpu_sc as plsc

_N = 8192
_LAT = 256
_DIM = 512
_NCB = 4096

_BN = 2048
_BK = 512
_NB = _N // _BN
_KB = _NCB // _BK

_NC = 2
_NS = 16
_NW = _NC * _NS
_BPW = _N // _NW
_CH = 128
_NCHUNK = _BPW // _CH


def _epilogue(mm, znorm, cnorm, iota_row, jjf):
    d = (znorm + cnorm) - 2.0 * mm
    lmin = jnp.min(d, axis=1, keepdims=True)
    cand = jnp.where(d == lmin, iota_row, float(_NCB))
    larg = jnp.min(cand, axis=1, keepdims=True) + jjf
    return lmin, larg


def _vq_tc_body(zr_ref, zi_ref, znorm_ref, cb_ref, cnorm_prev_ref,
                cnorm_cur_ref, iota_ref, idx_ref, dsum_ref, mm_ref,
                runmin_ref, runidx_ref, acc_ref):
    i = pl.program_id(0)
    j = pl.program_id(1)
    znorm = znorm_ref[...]
    iota_row = iota_ref[...]

    jjf = ((j - 1) * _BK).astype(jnp.float32)
    lmin, larg = _epilogue(mm_ref[...], znorm, cnorm_prev_ref[...],
                           iota_row, jjf)
    better = (lmin < runmin_ref[...]) | (j == 1)
    runmin_ref[...] = jnp.where(better, lmin, runmin_ref[...])
    runidx_ref[...] = jnp.where(better, larg, runidx_ref[...])

    cb = cb_ref[...]
    mm_ref[...] = (
        lax.dot_general(zr_ref[...], cb[:, :_LAT], (((1,), (1,)), ((), ())),
                        preferred_element_type=jnp.float32)
        + lax.dot_general(zi_ref[...], cb[:, _LAT:], (((1,), (1,)), ((), ())),
                          preferred_element_type=jnp.float32))

    @pl.when(j == _KB - 1)
    def _():
        lmin2, larg2 = _epilogue(mm_ref[...], znorm, cnorm_cur_ref[...],
                                 iota_row, jnp.float32(j * _BK))
        better2 = lmin2 < runmin_ref[...]
        fmin = jnp.where(better2, lmin2, runmin_ref[...])
        fidx = jnp.where(better2, larg2, runidx_ref[...]).astype(jnp.int32)
        idx_ref[...] = jnp.reshape(fidx, (_BN,))
        part = jnp.sum(fmin)
        prev = jnp.where(i == 0, 0.0, acc_ref[0])
        tot = prev + part
        acc_ref[0] = tot
        dsum_ref[...] = jnp.reshape(tot, (1, 1))


def _vq_argmin(z_r, z_i, znorm2d, cb, cnorm2d):
    idx2d, dsum = pl.pallas_call(
        _vq_tc_body,
        grid=(_NB, _KB),
        in_specs=[
            pl.BlockSpec((_BN, _LAT), lambda i, j: (i, 0)),
            pl.BlockSpec((_BN, _LAT), lambda i, j: (i, 0)),
            pl.BlockSpec((_BN, 1), lambda i, j: (i, 0)),
            pl.BlockSpec((_BK, _DIM), lambda i, j: (j, 0)),
            pl.BlockSpec((1, _BK), lambda i, j: (0, jnp.maximum(j - 1, 0))),
            pl.BlockSpec((1, _BK), lambda i, j: (0, j)),
            pl.BlockSpec((1, _BK), lambda i, j: (0, 0)),
        ],
        out_specs=[
            pl.BlockSpec((_BN,), lambda i, j: (i,)),
            pl.BlockSpec((1, 1), lambda i, j: (0, 0)),
        ],
        out_shape=[
            jax.ShapeDtypeStruct((_N,), jnp.int32),
            jax.ShapeDtypeStruct((1, 1), jnp.float32),
        ],
        scratch_shapes=[
            pltpu.VMEM((_BN, _BK), jnp.float32),
            pltpu.VMEM((_BN, 1), jnp.float32),
            pltpu.VMEM((_BN, 1), jnp.float32),
            pltpu.SMEM((1,), jnp.float32),
        ],
        compiler_params=pltpu.CompilerParams(
            dimension_semantics=("arbitrary", "arbitrary"),
        ),
    )(z_r, z_i, znorm2d, cb, cnorm2d, cnorm2d,
      jnp.arange(_BK, dtype=jnp.float32)[None, :])
    return idx2d, dsum[0, 0]


def _sc_gather_one(cb_r, cb_i, idx):
    mesh = plsc.VectorSubcoreMesh(core_axis_name="c", subcore_axis_name="s")

    @functools.partial(
        pl.kernel,
        out_type=[
            jax.ShapeDtypeStruct((_N, _LAT), jnp.float32),
            jax.ShapeDtypeStruct((_N, _LAT), jnp.float32),
        ],
        mesh=mesh,
        scratch_types=[
            pltpu.VMEM((_CH,), jnp.int32),
            pltpu.VMEM((_CH, _LAT), jnp.float32),
            pltpu.VMEM((_CH, _LAT), jnp.float32),
            pltpu.SemaphoreType.DMA,
        ],
    )
    def k(cbr_hbm, cbi_hbm, idx_hbm, outr_hbm, outi_hbm,
          idx_v, rows_r, rows_i, sem):
        wid = lax.axis_index("s") * _NC + lax.axis_index("c")
        base = wid * _BPW
        for c in range(_NCHUNK):
            off = base + c * _CH
            pltpu.sync_copy(idx_hbm.at[pl.ds(off, _CH)], idx_v)
            cp_r = pltpu.async_copy(cbr_hbm.at[idx_v], rows_r, sem)
            cp_i = pltpu.async_copy(cbi_hbm.at[idx_v], rows_i, sem)
            cp_r.wait()
            cp_i.wait()
            pltpu.sync_copy(rows_r, outr_hbm.at[pl.ds(off, _CH)])
            pltpu.sync_copy(rows_i, outi_hbm.at[pl.ds(off, _CH)])

    return k(cb_r, cb_i, idx)


def kernel(z_fast_real, z_fast_imag, z_slow_real, z_slow_imag, cb_syn, cb_sem):
    zf = jnp.concatenate([z_fast_real, z_fast_imag], axis=-1)
    zs = jnp.concatenate([z_slow_real, z_slow_imag], axis=-1)
    znf = jnp.sum(zf ** 2, axis=1)[:, None]
    zns = jnp.sum(zs ** 2, axis=1)[:, None]
    cns = jnp.sum(cb_syn ** 2, axis=1)[None, :]
    cnm = jnp.sum(cb_sem ** 2, axis=1)[None, :]
    idx_syn, dsum_syn = _vq_argmin(z_fast_real, z_fast_imag, znf, cb_syn, cns)
    rs_r, rs_i = _sc_gather_one(cb_syn[:, :_LAT], cb_syn[:, _LAT:], idx_syn)
    idx_sem, dsum_sem = _vq_argmin(z_slow_real, z_slow_imag, zns, cb_sem, cnm)
    rm_r, rm_i = _sc_gather_one(cb_sem[:, :_LAT], cb_sem[:, _LAT:], idx_sem)
    loss = 1.25 * (dsum_syn + dsum_sem) / (_N * _DIM)
    zq_syn = lax.complex(rs_r, rs_i)
    zq_sem = lax.complex(rm_r, rm_i)
    return (zq_syn, zq_sem, loss, idx_syn, idx_sem)
